# baseline (speedup 1.0000x reference)
import functools

import jax
import jax.numpy as jnp
from jax import lax
from jax.experimental import pallas as pl
from jax.experimental.pallas import tpu as pltpu
from jax.experimental.pallas import tpu_sc as plsc

N_NODES = 1000000
N_DIM = 16
BATCH = 16384

NC = 2
NS = 16
NW = NC * NS
BPW = BATCH // NW
ROWS = 2 * BPW
GROUPS = BPW // 16
CHUNK = 16

_mesh = plsc.VectorSubcoreMesh(core_axis_name="c", subcore_axis_name="s")


@functools.partial(
    pl.kernel,
    out_type=jax.ShapeDtypeStruct((BATCH,), jnp.float32),
    mesh=_mesh,
    compiler_params=pltpu.CompilerParams(
        needs_layout_passes=False, use_tc_tiling_on_sc=False),
    scratch_types=[
        pltpu.VMEM((ROWS,), jnp.int32),
        pltpu.VMEM((ROWS * N_DIM,), jnp.float32),
        pltpu.VMEM((16 * 16,), jnp.float32),
        pltpu.VMEM((BPW,), jnp.float32),
        pltpu.SemaphoreType.DMA,
        pltpu.SemaphoreType.DMA,
    ],
)
def _sc_inner(uv_hbm, tlin_hbm, out_hbm, idx_v, rows_v, wbuf, inner_v,
              sem_a, sem_b):
    wid = lax.axis_index("s") * NC + lax.axis_index("c")
    cp_u = pltpu.async_copy(
        uv_hbm.at[pl.ds(wid * BPW, BPW)], idx_v.at[pl.ds(0, BPW)], sem_a)
    cp_v = pltpu.async_copy(
        uv_hbm.at[pl.ds(BATCH + wid * BPW, BPW)],
        idx_v.at[pl.ds(BPW, BPW)], sem_b)
    cp_u.wait()
    cp_v.wait()

    def issue_chunk(c, sem):
        j0 = c * CHUNK
        nvec = idx_v[pl.ds(j0, CHUNK)] * N_DIM
        for jj in range(CHUNK):
            pltpu.async_copy(
                tlin_hbm.at[pl.ds(pl.multiple_of(nvec[jj], N_DIM), N_DIM)],
                rows_v.at[pl.ds((j0 + jj) * N_DIM, N_DIM)], sem)

    def drain_chunk(sem):
        pltpu.make_async_copy(
            tlin_hbm.at[pl.ds(0, CHUNK * N_DIM)],
            rows_v.at[pl.ds(0, CHUNK * N_DIM)], sem).wait()

    def pair_body(t, _):
        issue_chunk(2 * t, sem_a)
        issue_chunk(2 * t + 1, sem_b)
        drain_chunk(sem_a)
        drain_chunk(sem_b)
        return 0

    lax.fori_loop(0, ROWS // (2 * CHUNK), pair_body, 0)

    iv = lax.iota(jnp.int32, 16)

    def group_body(g, _):
        for k in range(16):
            j = g * 16 + k
            prod = (rows_v[pl.ds(j * N_DIM, N_DIM)] *
                    rows_v[pl.ds((BPW + j) * N_DIM, N_DIM)])
            plsc.store_scatter(wbuf, [iv * 16 + k], prod)
        acc = -wbuf[pl.ds(0, 16)]
        for d in range(1, N_DIM):
            acc = acc + wbuf[pl.ds(d * 16, 16)]
        inner_v[pl.ds(g * 16, 16)] = acc
        return 0

    lax.fori_loop(0, GROUPS, group_body, 0)
    pltpu.sync_copy(inner_v, out_hbm.at[pl.ds(wid * BPW, BPW)])


def _loss_body(bg_ref, inner_ref, lab_ref, out_ref):
    inner = inner_ref[...]
    x = jnp.maximum(-inner, 1.0 + 1e-7)
    dist = jnp.log(x + jnp.sqrt((x - 1.0) * (x + 1.0)))
    z = bg_ref[0] * dist - bg_ref[1]
    t = jnp.log1p(jnp.exp(-jnp.abs(z)))
    out_ref[...] = t + jnp.where(
        lab_ref[...] == 1, jnp.maximum(z, 0.0), jnp.maximum(-z, 0.0))


_tc_loss = pl.pallas_call(
    _loss_body,
    out_shape=jax.ShapeDtypeStruct((128, 128), jnp.float32),
    in_specs=[
        pl.BlockSpec(memory_space=pltpu.SMEM),
        pl.BlockSpec(memory_space=pltpu.VMEM),
        pl.BlockSpec(memory_space=pltpu.VMEM),
    ],
    out_specs=pl.BlockSpec(memory_space=pltpu.VMEM),
)


def kernel(pairs, labels, table, beta, gamma):
    pairs32 = pairs.astype(jnp.int32)
    uv = jnp.concatenate([pairs32[:, 0], pairs32[:, 1]])
    inner = _sc_inner(uv, table.reshape(-1))
    bg = jnp.stack([beta, gamma]).astype(jnp.float32)
    loss = _tc_loss(bg, inner.reshape(128, 128),
                    labels.astype(jnp.int32).reshape(128, 128))
    return loss.reshape(-1)


# speedup vs baseline: 1.0341x; 1.0341x over previous
import functools

import jax
import jax.numpy as jnp
from jax import lax
from jax.experimental import pallas as pl
from jax.experimental.pallas import tpu as pltpu
from jax.experimental.pallas import tpu_sc as plsc

N_NODES = 1000000
N_DIM = 16
BATCH = 16384

NC = 2
NS = 16
NW = NC * NS
BPW = BATCH // NW
ROWS = 2 * BPW
CHUNK = 128
GROUPS = BPW // 16

_mesh = plsc.VectorSubcoreMesh(core_axis_name="c", subcore_axis_name="s")


@functools.partial(
    pl.kernel,
    out_type=jax.ShapeDtypeStruct((BATCH,), jnp.float32),
    mesh=_mesh,
    compiler_params=pltpu.CompilerParams(
        needs_layout_passes=False, use_tc_tiling_on_sc=False),
    scratch_types=[
        pltpu.VMEM((ROWS,), jnp.int32),
        pltpu.VMEM((ROWS, N_DIM), jnp.float32),
        pltpu.VMEM((16 * 16,), jnp.float32),
        pltpu.VMEM((BPW,), jnp.float32),
        pltpu.SemaphoreType.DMA,
    ],
)
def _sc_inner(pairs_hbm, table_hbm, out_hbm, idx_v, rows_v, wbuf, inner_v,
              sem):
    wid = lax.axis_index("s") * NC + lax.axis_index("c")
    base = wid * ROWS
    pltpu.sync_copy(pairs_hbm.at[pl.ds(base, ROWS)], idx_v)
    copies = []
    for c in range(ROWS // CHUNK):
        copies.append(pltpu.async_copy(
            table_hbm.at[idx_v.at[pl.ds(c * CHUNK, CHUNK)]],
            rows_v.at[pl.ds(c * CHUNK, CHUNK)],
            sem,
        ))
    for cp in copies:
        cp.wait()

    iv = lax.iota(jnp.int32, 16)

    def group_body(g, _):
        for k in range(16):
            j = g * 16 + k
            prod = rows_v[2 * j] * rows_v[2 * j + 1]
            plsc.store_scatter(wbuf, [iv * 16 + k], prod)
        acc = -wbuf[pl.ds(0, 16)]
        for d in range(1, N_DIM):
            acc = acc + wbuf[pl.ds(d * 16, 16)]
        inner_v[pl.ds(g * 16, 16)] = acc
        return 0

    lax.fori_loop(0, GROUPS, group_body, 0)
    pltpu.sync_copy(inner_v, out_hbm.at[pl.ds(wid * BPW, BPW)])


def _loss_body(bg_ref, inner_ref, lab_ref, out_ref):
    inner = inner_ref[...]
    x = jnp.maximum(-inner, 1.0 + 1e-7)
    dist = jnp.log(x + jnp.sqrt((x - 1.0) * (x + 1.0)))
    z = bg_ref[0] * dist - bg_ref[1]
    t = jnp.log1p(jnp.exp(-jnp.abs(z)))
    out_ref[...] = t + jnp.where(
        lab_ref[...] == 1, jnp.maximum(z, 0.0), jnp.maximum(-z, 0.0))


_tc_loss = pl.pallas_call(
    _loss_body,
    out_shape=jax.ShapeDtypeStruct((128, 128), jnp.float32),
    in_specs=[
        pl.BlockSpec(memory_space=pltpu.SMEM),
        pl.BlockSpec(memory_space=pltpu.VMEM),
        pl.BlockSpec(memory_space=pltpu.VMEM),
    ],
    out_specs=pl.BlockSpec(memory_space=pltpu.VMEM),
)


def kernel(pairs, labels, table, beta, gamma):
    pairs_flat = pairs.astype(jnp.int32).reshape(-1)
    inner = _sc_inner(pairs_flat, table)
    bg = jnp.stack([beta, gamma]).astype(jnp.float32)
    loss = _tc_loss(bg, inner.reshape(128, 128),
                    labels.astype(jnp.int32).reshape(128, 128))
    return loss.reshape(-1)
